# baseline (speedup 1.0000x reference)
import functools

import jax
import jax.numpy as jnp
from jax import lax
from jax.experimental import pallas as pl
from jax.experimental.pallas import tpu as pltpu
from jax.experimental.pallas import tpu_sc as plsc

_VOCAB = 1000000
_EMB = 64
_OUT = 128
_B = 4096
_L = 200

_NC = 2
_NS = 16
_NW = _NC * _NS
_CW = 100
_CHUNKS = _B * _L // _CW
_CH_PER_W = _CHUNKS // _NW
_ROWS_PER_W = _B // _NW
_NLANE = 16
_NV = _EMB // _NLANE


def _sc_body(x_hbm, emb_hbm, out_hbm, idx_v, gbuf, obuf, sem0, sem1):
    c = lax.axis_index("c")
    s = lax.axis_index("s")
    w = s * _NC + c

    pltpu.sync_copy(x_hbm.at[pl.ds(w * _CH_PER_W, _CH_PER_W)], idx_v)

    pltpu.async_copy(emb_hbm.at[idx_v.at[0]], gbuf.at[0], sem0)
    pltpu.async_copy(emb_hbm.at[idx_v.at[1]], gbuf.at[1], sem1)

    zero = jnp.zeros((_NLANE,), jnp.float32)

    def reduce_buf(bi, accs):
        def rbody(r, accs):
            accs = list(accs)
            base = r * 5
            for u in range(5):
                row = base + u
                for v in range(_NV):
                    accs[v] = accs[v] + gbuf[bi, row, pl.ds(v * _NLANE, _NLANE)]
            return tuple(accs)
        return lax.fori_loop(0, _CW // 5, rbody, accs)

    @pl.loop(0, _ROWS_PER_W)
    def _(i):
        j0 = 2 * i
        pltpu.make_async_copy(emb_hbm.at[idx_v.at[0]], gbuf.at[0], sem0).wait()
        a = reduce_buf(0, (zero,) * _NV)

        @pl.when(i < _ROWS_PER_W - 1)
        def _():
            pltpu.async_copy(emb_hbm.at[idx_v.at[j0 + 2]], gbuf.at[0], sem0)

        pltpu.make_async_copy(emb_hbm.at[idx_v.at[1]], gbuf.at[1], sem1).wait()
        a = reduce_buf(1, a)

        @pl.when(i < _ROWS_PER_W - 1)
        def _():
            pltpu.async_copy(emb_hbm.at[idx_v.at[j0 + 3]], gbuf.at[1], sem1)

        for v in range(_NV):
            obuf[i, pl.ds(v * _NLANE, _NLANE)] = a[v]

    pltpu.sync_copy(obuf, out_hbm.at[pl.ds(w * _ROWS_PER_W, _ROWS_PER_W)])


def _sc_pool(xr, emb):
    mesh = plsc.VectorSubcoreMesh(core_axis_name="c", subcore_axis_name="s")
    return pl.kernel(
        _sc_body,
        out_type=jax.ShapeDtypeStruct((_B, _EMB), jnp.float32),
        mesh=mesh,
        compiler_params=pltpu.CompilerParams(use_tc_tiling_on_sc=False),
        scratch_types=[
            pltpu.VMEM((_CH_PER_W, _CW), jnp.int32),
            pltpu.VMEM((2, _CW, _EMB), jnp.float32),
            pltpu.VMEM((_ROWS_PER_W, _EMB), jnp.float32),
            pltpu.SemaphoreType.DMA,
            pltpu.SemaphoreType.DMA,
        ],
    )(xr, emb)


def _head_body(s_ref, wt_ref, b_ref, o_ref):
    p = s_ref[...] * (1.0 / _L)
    logits = jnp.dot(p, wt_ref[...], preferred_element_type=jnp.float32)
    logits = logits + b_ref[...]
    m = jnp.max(logits, axis=-1, keepdims=True)
    e = jnp.exp(logits - m)
    lse = jnp.log(jnp.sum(e, axis=-1, keepdims=True))
    o_ref[...] = logits - m - lse


def _head(sums, wt, b2):
    blk = 1024
    return pl.pallas_call(
        _head_body,
        grid=(_B // blk,),
        in_specs=[
            pl.BlockSpec((blk, _EMB), lambda i: (i, 0)),
            pl.BlockSpec((_EMB, _OUT), lambda i: (0, 0)),
            pl.BlockSpec((1, _OUT), lambda i: (0, 0)),
        ],
        out_specs=pl.BlockSpec((blk, _OUT), lambda i: (i, 0)),
        out_shape=jax.ShapeDtypeStruct((_B, _OUT), jnp.float32),
    )(sums, wt, b2)


@jax.jit
def kernel(x, emb, W, b):
    xr = x.reshape(_CHUNKS, _CW)
    sums = _sc_pool(xr, emb)
    return _head(sums, W.T, b.reshape(1, _OUT))


# speedup vs baseline: 1.0685x; 1.0685x over previous
import functools

import jax
import jax.numpy as jnp
from jax import lax
from jax.experimental import pallas as pl
from jax.experimental.pallas import tpu as pltpu
from jax.experimental.pallas import tpu_sc as plsc

_VOCAB = 1000000
_EMB = 64
_OUT = 128
_B = 4096
_L = 200

_NC = 2
_NS = 16
_NW = _NC * _NS
_CW = 100
_CHUNKS = _B * _L // _CW
_CH_PER_W = _CHUNKS // _NW
_ROWS_PER_W = _B // _NW
_NLANE = 16
_NV = _EMB // _NLANE


def _sc_body(x_hbm, emb_hbm, out_hbm, idx_v, gbuf, obuf, sem0, sem1):
    c = lax.axis_index("c")
    s = lax.axis_index("s")
    w = s * _NC + c

    pltpu.sync_copy(x_hbm.at[pl.ds(w * _CH_PER_W, _CH_PER_W)], idx_v)

    pltpu.async_copy(emb_hbm.at[idx_v.at[0]], gbuf.at[0], sem0)
    pltpu.async_copy(emb_hbm.at[idx_v.at[1]], gbuf.at[1], sem1)

    zero = jnp.zeros((_NLANE,), jnp.float32)

    def reduce_buf(bi, accs):
        def rbody(r, accs):
            accs = list(accs)
            base = r * 5
            for u in range(5):
                row = base + u
                for v in range(_NV):
                    accs[v] = accs[v] + gbuf[bi, row, pl.ds(v * _NLANE, _NLANE)]
            return tuple(accs)
        return lax.fori_loop(0, _CW // 5, rbody, accs)

    @pl.loop(0, _ROWS_PER_W)
    def _(i):
        j0 = 2 * i
        pltpu.make_async_copy(emb_hbm.at[idx_v.at[0]], gbuf.at[0], sem0).wait()
        a = reduce_buf(0, (zero,) * _NV)

        @pl.when(i < _ROWS_PER_W - 1)
        def _():
            pltpu.async_copy(emb_hbm.at[idx_v.at[j0 + 2]], gbuf.at[0], sem0)

        pltpu.make_async_copy(emb_hbm.at[idx_v.at[1]], gbuf.at[1], sem1).wait()
        a = reduce_buf(1, a)

        @pl.when(i < _ROWS_PER_W - 1)
        def _():
            pltpu.async_copy(emb_hbm.at[idx_v.at[j0 + 3]], gbuf.at[1], sem1)

        for v in range(_NV):
            obuf[i, pl.ds(v * _NLANE, _NLANE)] = a[v]

    pltpu.sync_copy(obuf, out_hbm.at[pl.ds(w * _ROWS_PER_W, _ROWS_PER_W)])


def _sc_pool(xr, emb):
    mesh = plsc.VectorSubcoreMesh(core_axis_name="c", subcore_axis_name="s")
    return pl.kernel(
        _sc_body,
        out_type=jax.ShapeDtypeStruct((_B, _EMB), jnp.float32),
        mesh=mesh,
        compiler_params=pltpu.CompilerParams(use_tc_tiling_on_sc=False),
        scratch_types=[
            pltpu.VMEM((_CH_PER_W, _CW), jnp.int32),
            pltpu.VMEM((2, _CW, _EMB), jnp.float32),
            pltpu.VMEM((_ROWS_PER_W, _EMB), jnp.float32),
            pltpu.SemaphoreType.DMA,
            pltpu.SemaphoreType.DMA,
        ],
    )(xr, emb)


_TRB = 2048


def _tr_body(t_ref, o_ref):
    t = t_ref[...].T.reshape(_TRB // 2, 2, _EMB)
    o_ref[...] = jnp.concatenate([t[:, 0, :], t[:, 1, :]], axis=1)


def _tc_linearize(embT):
    return pl.pallas_call(
        _tr_body,
        grid=(pl.cdiv(_VOCAB, _TRB),),
        in_specs=[pl.BlockSpec((_EMB, _TRB), lambda i: (0, i))],
        out_specs=pl.BlockSpec((_TRB // 2, 2 * _EMB), lambda i: (i, 0)),
        out_shape=jax.ShapeDtypeStruct((_VOCAB // 2, 2 * _EMB), jnp.float32),
    )(embT)


def _head_body(s_ref, wt_ref, b_ref, o_ref):
    p = s_ref[...] * (1.0 / _L)
    logits = jnp.dot(p, wt_ref[...], preferred_element_type=jnp.float32)
    logits = logits + b_ref[...]
    m = jnp.max(logits, axis=-1, keepdims=True)
    e = jnp.exp(logits - m)
    lse = jnp.log(jnp.sum(e, axis=-1, keepdims=True))
    o_ref[...] = logits - m - lse


def _head(sums, wt, b2):
    blk = 1024
    return pl.pallas_call(
        _head_body,
        grid=(_B // blk,),
        in_specs=[
            pl.BlockSpec((blk, _EMB), lambda i: (i, 0)),
            pl.BlockSpec((_EMB, _OUT), lambda i: (0, 0)),
            pl.BlockSpec((1, _OUT), lambda i: (0, 0)),
        ],
        out_specs=pl.BlockSpec((blk, _OUT), lambda i: (i, 0)),
        out_shape=jax.ShapeDtypeStruct((_B, _OUT), jnp.float32),
    )(sums, wt, b2)


@jax.jit
def kernel(x, emb, W, b):
    xr = x.reshape(_CHUNKS, _CW)
    emb_lin = _tc_linearize(emb.T).reshape(_VOCAB, _EMB)
    sums = _sc_pool(xr, emb_lin)
    return _head(sums, W.T, b.reshape(1, _OUT))


# speedup vs baseline: 1.4913x; 1.3956x over previous
import functools

import jax
import jax.numpy as jnp
from jax import lax
from jax.experimental import pallas as pl
from jax.experimental.pallas import tpu as pltpu
from jax.experimental.pallas import tpu_sc as plsc

_VOCAB = 1000000
_EMB = 64
_OUT = 128
_B = 4096
_L = 200

_NC = 2
_NS = 16
_NW = _NC * _NS
_CW = 100
_CHUNKS = _B * _L // _CW
_CH_PER_W = _CHUNKS // _NW
_ROWS_PER_W = _B // _NW
_NLANE = 16
_NV = _EMB // _NLANE


def _sc_body(x_hbm, emb_hbm, out_hbm, idx_v, gbuf, obuf, sem0, sem1):
    c = lax.axis_index("c")
    s = lax.axis_index("s")
    w = s * _NC + c

    pltpu.sync_copy(x_hbm.at[pl.ds(w * _CH_PER_W, _CH_PER_W)], idx_v)

    pltpu.async_copy(emb_hbm.at[idx_v.at[0]], gbuf.at[0], sem0)
    pltpu.async_copy(emb_hbm.at[idx_v.at[1]], gbuf.at[1], sem1)

    zero = jnp.zeros((_NLANE,), jnp.float32)

    def reduce_buf(bi, accs):
        def rbody(r, accs):
            accs = list(accs)
            base = r * 5
            for u in range(5):
                row = base + u
                for v in range(_NV):
                    accs[v] = accs[v] + gbuf[bi, row, pl.ds(v * _NLANE, _NLANE)]
            return tuple(accs)
        return lax.fori_loop(0, _CW // 5, rbody, accs)

    @pl.loop(0, _ROWS_PER_W)
    def _(i):
        j0 = 2 * i
        pltpu.make_async_copy(emb_hbm.at[idx_v.at[0]], gbuf.at[0], sem0).wait()
        a = reduce_buf(0, (zero,) * _NV)

        @pl.when(i < _ROWS_PER_W - 1)
        def _():
            pltpu.async_copy(emb_hbm.at[idx_v.at[j0 + 2]], gbuf.at[0], sem0)

        pltpu.make_async_copy(emb_hbm.at[idx_v.at[1]], gbuf.at[1], sem1).wait()
        a = reduce_buf(1, a)

        @pl.when(i < _ROWS_PER_W - 1)
        def _():
            pltpu.async_copy(emb_hbm.at[idx_v.at[j0 + 3]], gbuf.at[1], sem1)

        for v in range(_NV):
            obuf[i, pl.ds(v * _NLANE, _NLANE)] = a[v]

    pltpu.sync_copy(obuf, out_hbm.at[pl.ds(w * _ROWS_PER_W, _ROWS_PER_W)])


def _sc_pool(xr, emb):
    mesh = plsc.VectorSubcoreMesh(core_axis_name="c", subcore_axis_name="s")
    return pl.kernel(
        _sc_body,
        out_type=jax.ShapeDtypeStruct((_B, _EMB), jnp.float32),
        mesh=mesh,
        compiler_params=pltpu.CompilerParams(use_tc_tiling_on_sc=False),
        scratch_types=[
            pltpu.VMEM((_CH_PER_W, _CW), jnp.int32),
            pltpu.VMEM((2, _CW, _EMB), jnp.float32),
            pltpu.VMEM((_ROWS_PER_W, _EMB), jnp.float32),
            pltpu.SemaphoreType.DMA,
            pltpu.SemaphoreType.DMA,
        ],
    )(xr, emb)


_TRB = 4096
_SH = (_TRB // 2).bit_length() - 1
_NBLK = (_VOCAB + _TRB - 1) // _TRB
_VROWS = _NBLK * _TRB


def _tr_body(t_ref, o_ref):
    t = t_ref[...].T
    o_ref[...] = jnp.concatenate([t[: _TRB // 2], t[_TRB // 2 :]], axis=1)


def _tc_linearize(embT):
    return pl.pallas_call(
        _tr_body,
        grid=(_NBLK,),
        in_specs=[pl.BlockSpec((_EMB, _TRB), lambda i: (0, i))],
        out_specs=pl.BlockSpec((_TRB // 2, 2 * _EMB), lambda i: (i, 0)),
        out_shape=jax.ShapeDtypeStruct((_VROWS // 2, 2 * _EMB), jnp.float32),
    )(embT)


def _head_body(s_ref, wt_ref, b_ref, o_ref):
    p = s_ref[...] * (1.0 / _L)
    logits = jnp.dot(p, wt_ref[...], preferred_element_type=jnp.float32)
    logits = logits + b_ref[...]
    m = jnp.max(logits, axis=-1, keepdims=True)
    e = jnp.exp(logits - m)
    lse = jnp.log(jnp.sum(e, axis=-1, keepdims=True))
    o_ref[...] = logits - m - lse


def _head(sums, wt, b2):
    blk = 1024
    return pl.pallas_call(
        _head_body,
        grid=(_B // blk,),
        in_specs=[
            pl.BlockSpec((blk, _EMB), lambda i: (i, 0)),
            pl.BlockSpec((_EMB, _OUT), lambda i: (0, 0)),
            pl.BlockSpec((1, _OUT), lambda i: (0, 0)),
        ],
        out_specs=pl.BlockSpec((blk, _OUT), lambda i: (i, 0)),
        out_shape=jax.ShapeDtypeStruct((_B, _OUT), jnp.float32),
    )(sums, wt, b2)


@jax.jit
def kernel(x, emb, W, b):
    xf = (x & -_TRB) | ((x & (_TRB // 2 - 1)) << 1) | ((x >> _SH) & 1)
    xr = xf.reshape(_CHUNKS, _CW)
    emb_lin = _tc_linearize(emb.T).reshape(_VROWS, _EMB)
    sums = _sc_pool(xr, emb_lin)
    return _head(sums, W.T, b.reshape(1, _OUT))


# speedup vs baseline: 2.0014x; 1.3421x over previous
import functools

import jax
import jax.numpy as jnp
from jax import lax
from jax.experimental import pallas as pl
from jax.experimental.pallas import tpu as pltpu
from jax.experimental.pallas import tpu_sc as plsc

_VOCAB = 1000000
_EMB = 64
_OUT = 128
_B = 4096
_L = 200

_NC = 2
_NS = 16
_NW = _NC * _NS
_CW = 100
_CHUNKS = _B * _L // _CW
_CH_PER_W = _CHUNKS // _NW
_ROWS_PER_W = _B // _NW
_NLANE = 16
_NV = _EMB // _NLANE


def _sc_body(x_hbm, emb_hbm, out_hbm, idx_v, gbuf, obuf, sem0, sem1, sem2, sem3):
    c = lax.axis_index("c")
    s = lax.axis_index("s")
    w = s * _NC + c

    pltpu.sync_copy(x_hbm.at[pl.ds(w * _CH_PER_W, _CH_PER_W)], idx_v)

    pltpu.async_copy(emb_hbm.at[idx_v.at[0]], gbuf.at[0], sem0)
    pltpu.async_copy(emb_hbm.at[idx_v.at[1]], gbuf.at[1], sem1)
    pltpu.async_copy(emb_hbm.at[idx_v.at[2]], gbuf.at[2], sem2)
    pltpu.async_copy(emb_hbm.at[idx_v.at[3]], gbuf.at[3], sem3)

    sems = (sem0, sem1, sem2, sem3)
    zero = jnp.zeros((_NLANE,), jnp.float32)

    def reduce_buf(bi, accs):
        def rbody(r, accs):
            accs = list(accs)
            base = r * 5
            for u in range(5):
                row = base + u
                for v in range(_NV):
                    accs[v] = accs[v] + gbuf[bi, row, pl.ds(v * _NLANE, _NLANE)]
            return tuple(accs)
        return lax.fori_loop(0, _CW // 5, rbody, accs)

    @pl.loop(0, _ROWS_PER_W // 2)
    def _(i):
        j0 = 4 * i
        for half in range(2):
            a = (zero,) * _NV
            for q in range(2):
                b = 2 * half + q
                pltpu.make_async_copy(
                    emb_hbm.at[idx_v.at[0]], gbuf.at[b], sems[b]).wait()
                a = reduce_buf(b, a)

                @pl.when(j0 + b + 4 < _CH_PER_W)
                def _():
                    pltpu.async_copy(
                        emb_hbm.at[idx_v.at[j0 + b + 4]], gbuf.at[b], sems[b])

            row = 2 * i + half
            for v in range(_NV):
                obuf[row, pl.ds(v * _NLANE, _NLANE)] = a[v]

    pltpu.sync_copy(obuf, out_hbm.at[pl.ds(w * _ROWS_PER_W, _ROWS_PER_W)])


def _sc_pool(xr, emb):
    mesh = plsc.VectorSubcoreMesh(core_axis_name="c", subcore_axis_name="s")
    return pl.kernel(
        _sc_body,
        out_type=jax.ShapeDtypeStruct((_B, _EMB), jnp.float32),
        mesh=mesh,
        compiler_params=pltpu.CompilerParams(use_tc_tiling_on_sc=False),
        scratch_types=[
            pltpu.VMEM((_CH_PER_W, _CW), jnp.int32),
            pltpu.VMEM((4, _CW, _EMB), jnp.float32),
            pltpu.VMEM((_ROWS_PER_W, _EMB), jnp.float32),
            pltpu.SemaphoreType.DMA,
            pltpu.SemaphoreType.DMA,
            pltpu.SemaphoreType.DMA,
            pltpu.SemaphoreType.DMA,
        ],
    )(xr, emb)


_TRB = 8192
_SH = (_TRB // 2).bit_length() - 1
_NBLK = (_VOCAB + _TRB - 1) // _TRB
_VROWS = _NBLK * _TRB


def _tr_body(t_ref, o_ref):
    t = t_ref[...].T
    o_ref[...] = jnp.concatenate([t[: _TRB // 2], t[_TRB // 2 :]], axis=1)


def _tc_linearize(embT):
    return pl.pallas_call(
        _tr_body,
        grid=(_NBLK,),
        in_specs=[pl.BlockSpec((_EMB, _TRB), lambda i: (0, i))],
        out_specs=pl.BlockSpec((_TRB // 2, 2 * _EMB), lambda i: (i, 0)),
        out_shape=jax.ShapeDtypeStruct((_VROWS // 2, 2 * _EMB), jnp.float32),
    )(embT)


def _head_body(s_ref, wt_ref, b_ref, o_ref):
    p = s_ref[...] * (1.0 / _L)
    logits = jnp.dot(p, wt_ref[...], preferred_element_type=jnp.float32)
    logits = logits + b_ref[...]
    m = jnp.max(logits, axis=-1, keepdims=True)
    e = jnp.exp(logits - m)
    lse = jnp.log(jnp.sum(e, axis=-1, keepdims=True))
    o_ref[...] = logits - m - lse


def _head(sums, wt, b2):
    blk = 1024
    return pl.pallas_call(
        _head_body,
        grid=(_B // blk,),
        in_specs=[
            pl.BlockSpec((blk, _EMB), lambda i: (i, 0)),
            pl.BlockSpec((_EMB, _OUT), lambda i: (0, 0)),
            pl.BlockSpec((1, _OUT), lambda i: (0, 0)),
        ],
        out_specs=pl.BlockSpec((blk, _OUT), lambda i: (i, 0)),
        out_shape=jax.ShapeDtypeStruct((_B, _OUT), jnp.float32),
    )(sums, wt, b2)


@jax.jit
def kernel(x, emb, W, b):
    xf = (x & -_TRB) | ((x & (_TRB // 2 - 1)) << 1) | ((x >> _SH) & 1)
    xr = xf.reshape(_CHUNKS, _CW)
    emb_lin = _tc_linearize(emb.T).reshape(_VROWS, _EMB)
    sums = _sc_pool(xr, emb_lin)
    return _head(sums, W.T, b.reshape(1, _OUT))


# speedup vs baseline: 2.8437x; 1.4209x over previous
import functools

import jax
import jax.numpy as jnp
from jax import lax
from jax.experimental import pallas as pl
from jax.experimental.pallas import tpu as pltpu
from jax.experimental.pallas import tpu_sc as plsc

_VOCAB = 1000000
_EMB = 64
_OUT = 128
_B = 4096
_L = 200

_NC = 2
_NS = 16
_NW = _NC * _NS
_CW = 100
_CHUNKS = _B * _L // _CW
_CH_PER_W = _CHUNKS // _NW
_ROWS_PER_W = _B // _NW
_NLANE = 16
_NV = _EMB // _NLANE


_NBUF = 8


def _sc_body(x_hbm, emb_hbm, out_hbm, idx_v, gbuf, obuf, *sems):
    c = lax.axis_index("c")
    s = lax.axis_index("s")
    w = s * _NC + c

    pltpu.sync_copy(x_hbm.at[pl.ds(w * _CH_PER_W, _CH_PER_W)], idx_v)

    for b in range(_NBUF):
        pltpu.async_copy(emb_hbm.at[idx_v.at[b]], gbuf.at[b], sems[b])

    zero = jnp.zeros((_NLANE,), jnp.float32)

    def reduce_buf(bi, accs):
        def rbody(r, accs):
            accs = list(accs)
            base = r * 5
            for u in range(5):
                row = base + u
                for v in range(_NV):
                    accs[v] = accs[v] + gbuf[bi, row, pl.ds(v * _NLANE, _NLANE)]
            return tuple(accs)
        return lax.fori_loop(0, _CW // 5, rbody, accs)

    @pl.loop(0, _ROWS_PER_W // (_NBUF // 2))
    def _(i):
        j0 = _NBUF * i
        for half in range(_NBUF // 2):
            a = (zero,) * _NV
            for q in range(2):
                b = 2 * half + q
                pltpu.make_async_copy(
                    emb_hbm.at[idx_v.at[0]], gbuf.at[b], sems[b]).wait()
                a = reduce_buf(b, a)

                @pl.when(j0 + b + _NBUF < _CH_PER_W)
                def _():
                    pltpu.async_copy(
                        emb_hbm.at[idx_v.at[j0 + b + _NBUF]], gbuf.at[b], sems[b])

            row = (_NBUF // 2) * i + half
            for v in range(_NV):
                obuf[row, pl.ds(v * _NLANE, _NLANE)] = a[v]

    pltpu.sync_copy(obuf, out_hbm.at[pl.ds(w * _ROWS_PER_W, _ROWS_PER_W)])


def _sc_pool(xr, emb):
    mesh = plsc.VectorSubcoreMesh(core_axis_name="c", subcore_axis_name="s")
    return pl.kernel(
        _sc_body,
        out_type=jax.ShapeDtypeStruct((_B, _EMB), jnp.float32),
        mesh=mesh,
        compiler_params=pltpu.CompilerParams(use_tc_tiling_on_sc=False),
        scratch_types=[
            pltpu.VMEM((_CH_PER_W, _CW), jnp.int32),
            pltpu.VMEM((_NBUF, _CW, _EMB), jnp.float32),
            pltpu.VMEM((_ROWS_PER_W, _EMB), jnp.float32),
        ] + [pltpu.SemaphoreType.DMA] * _NBUF + [
        ],
    )(xr, emb)


_TRB = 32768
_SH = (_TRB // 2).bit_length() - 1
_NBLK = (_VOCAB + _TRB - 1) // _TRB
_VROWS = _NBLK * _TRB


def _tr_body(t_ref, o_ref):
    x2 = jnp.concatenate(
        [t_ref[:, : _TRB // 2], t_ref[:, _TRB // 2 :]], axis=0)
    o_ref[...] = x2.T


def _tc_linearize(embT):
    return pl.pallas_call(
        _tr_body,
        grid=(_NBLK,),
        in_specs=[pl.BlockSpec((_EMB, _TRB), lambda i: (0, i))],
        out_specs=pl.BlockSpec((_TRB // 2, 2 * _EMB), lambda i: (i, 0)),
        out_shape=jax.ShapeDtypeStruct((_VROWS // 2, 2 * _EMB), jnp.float32),
    )(embT)


def _head_body(s_ref, wt_ref, b_ref, o_ref):
    p = s_ref[...] * (1.0 / _L)
    logits = jnp.dot(p, wt_ref[...], preferred_element_type=jnp.float32)
    logits = logits + b_ref[...]
    m = jnp.max(logits, axis=-1, keepdims=True)
    e = jnp.exp(logits - m)
    lse = jnp.log(jnp.sum(e, axis=-1, keepdims=True))
    o_ref[...] = logits - m - lse


def _head(sums, wt, b2):
    blk = 1024
    return pl.pallas_call(
        _head_body,
        grid=(_B // blk,),
        in_specs=[
            pl.BlockSpec((blk, _EMB), lambda i: (i, 0)),
            pl.BlockSpec((_EMB, _OUT), lambda i: (0, 0)),
            pl.BlockSpec((1, _OUT), lambda i: (0, 0)),
        ],
        out_specs=pl.BlockSpec((blk, _OUT), lambda i: (i, 0)),
        out_shape=jax.ShapeDtypeStruct((_B, _OUT), jnp.float32),
    )(sums, wt, b2)


@jax.jit
def kernel(x, emb, W, b):
    xf = (x & -_TRB) | ((x & (_TRB // 2 - 1)) << 1) | ((x >> _SH) & 1)
    xr = xf.reshape(_CHUNKS, _CW)
    emb_lin = _tc_linearize(emb.T).reshape(_VROWS, _EMB)
    sums = _sc_pool(xr, emb_lin)
    return _head(sums, W.T, b.reshape(1, _OUT))
